# baseline (speedup 1.0000x reference)
import jax
import jax.numpy as jnp
from jax import lax
from jax.experimental import pallas as pl
from jax.experimental.pallas import tpu as pltpu
from jax.experimental.pallas import tpu_sc as plsc

BATCH = 128
MEM = 4096
VAL = 64
SC_B = 16
TC_B = BATCH - SC_B
B_CH = 2
NCH = TC_B // B_CH
NBUF = 7
LANES = 16
N_WORKERS = 32
ROWS_PW = SC_B * VAL // N_WORKERS


def _tc_kernel(mem_hbm, w_ref, vt_ref, out_hbm, in_buf, out_buf, in_sems, out_sems):
    def in_copy(c, slot):
        return pltpu.make_async_copy(
            mem_hbm.at[pl.ds(c * B_CH, B_CH)],
            in_buf.at[slot],
            in_sems.at[slot],
        )

    def out_copy(c, slot):
        return pltpu.make_async_copy(
            out_buf.at[slot],
            out_hbm.at[pl.ds(c * B_CH, B_CH)],
            out_sems.at[slot],
        )

    for c in range(NBUF):
        in_copy(c, c).start()

    for c in range(NCH):
        slot = c % NBUF
        in_copy(c, slot).wait()
        if c >= NBUF:
            out_copy(c - NBUF, slot).wait()
        wb = w_ref[pl.ds(c * B_CH, B_CH), :][:, None, :]
        vb = vt_ref[:, pl.ds(c * B_CH, B_CH)].T[:, :, None]
        out_buf[slot] = in_buf[slot] + wb * vb
        out_copy(c, slot).start()
        if c + NBUF < NCH:
            in_copy(c + NBUF, slot).start()

    for c in range(NCH - NBUF, NCH):
        out_copy(c, c % NBUF).wait()


def _sc_body(mem_hbm, w_hbm, vb_hbm, out_hbm, w_buf, row_buf, out_row, vv_buf):
    wid = lax.axis_index("s") * 2 + lax.axis_index("c")
    b_loc = wid // (VAL // ROWS_PW)
    j0 = (wid % (VAL // ROWS_PW)) * ROWS_PW
    pltpu.sync_copy(w_hbm.at[TC_B + b_loc], w_buf)

    @pl.loop(0, ROWS_PW)
    def _row(r):
        j = j0 + r
        pltpu.sync_copy(mem_hbm.at[TC_B + b_loc, j], row_buf)
        pltpu.sync_copy(vb_hbm.at[b_loc * VAL + j], vv_buf)

        @pl.loop(0, MEM // LANES)
        def _vec(i):
            k = i * LANES
            out_row[pl.ds(k, LANES)] = (
                row_buf[pl.ds(k, LANES)]
                + w_buf[pl.ds(k, LANES)] * vv_buf[...]
            )

        pltpu.sync_copy(out_row, out_hbm.at[b_loc, j])


def kernel(memory, w, v):
    mem_t = memory.transpose(0, 2, 1)
    vt = v.T
    vb = jnp.broadcast_to(
        v[TC_B:].reshape(SC_B * VAL, 1), (SC_B * VAL, LANES)
    )

    tc_out = pl.pallas_call(
        _tc_kernel,
        in_specs=[
            pl.BlockSpec(memory_space=pltpu.MemorySpace.HBM),
            pl.BlockSpec(memory_space=pltpu.MemorySpace.VMEM),
            pl.BlockSpec(memory_space=pltpu.MemorySpace.VMEM),
        ],
        out_specs=pl.BlockSpec(memory_space=pltpu.MemorySpace.HBM),
        out_shape=jax.ShapeDtypeStruct((TC_B, VAL, MEM), memory.dtype),
        scratch_shapes=[
            pltpu.VMEM((NBUF, B_CH, VAL, MEM), jnp.float32),
            pltpu.VMEM((NBUF, B_CH, VAL, MEM), jnp.float32),
            pltpu.SemaphoreType.DMA((NBUF,)),
            pltpu.SemaphoreType.DMA((NBUF,)),
        ],
    )(mem_t, w, vt)

    sc_kernel = pl.kernel(
        _sc_body,
        out_type=jax.ShapeDtypeStruct((SC_B, VAL, MEM), jnp.float32),
        mesh=plsc.VectorSubcoreMesh(core_axis_name="c", subcore_axis_name="s"),
        scratch_types=[
            pltpu.VMEM((MEM,), jnp.float32),
            pltpu.VMEM((MEM,), jnp.float32),
            pltpu.VMEM((MEM,), jnp.float32),
            pltpu.VMEM((LANES,), jnp.float32),
        ],
    )
    sc_out = sc_kernel(mem_t, w, vb)

    out_t = jnp.concatenate([tc_out, sc_out], axis=0)
    return out_t.transpose(0, 2, 1)


# speedup vs baseline: 2.7830x; 2.7830x over previous
import jax
import jax.numpy as jnp
from jax.experimental import pallas as pl
from jax.experimental.pallas import tpu as pltpu

BATCH = 128
MEM = 4096
VAL = 64
B_CH = 4
NCH = BATCH // B_CH
NBUF = 6


def _update_kernel(mem_hbm, w_ref, vt_ref, out_hbm, in_buf, out_buf, in_sems, out_sems):
    def in_copy(c, slot):
        return pltpu.make_async_copy(
            mem_hbm.at[pl.ds(c * B_CH, B_CH)],
            in_buf.at[slot],
            in_sems.at[slot],
        )

    def out_copy(c, slot):
        return pltpu.make_async_copy(
            out_buf.at[slot],
            out_hbm.at[pl.ds(c * B_CH, B_CH)],
            out_sems.at[slot],
        )

    for c in range(NBUF):
        in_copy(c, c).start()

    for c in range(NCH):
        slot = c % NBUF
        in_copy(c, slot).wait()
        if c >= NBUF:
            out_copy(c - NBUF, slot).wait()
        wb = w_ref[pl.ds(c * B_CH, B_CH), :][:, None, :]
        vb = vt_ref[:, pl.ds(c * B_CH, B_CH)].T[:, :, None]
        out_buf[slot] = in_buf[slot] + wb * vb
        out_copy(c, slot).start()
        if c + NBUF < NCH:
            in_copy(c + NBUF, slot).start()

    for c in range(NCH - NBUF, NCH):
        out_copy(c, c % NBUF).wait()


def kernel(memory, w, v):
    mem_t = memory.transpose(0, 2, 1)
    vt = v.T
    out_t = pl.pallas_call(
        _update_kernel,
        in_specs=[
            pl.BlockSpec(memory_space=pltpu.MemorySpace.HBM),
            pl.BlockSpec(memory_space=pltpu.MemorySpace.VMEM),
            pl.BlockSpec(memory_space=pltpu.MemorySpace.VMEM),
        ],
        out_specs=pl.BlockSpec(memory_space=pltpu.MemorySpace.HBM),
        out_shape=jax.ShapeDtypeStruct((BATCH, VAL, MEM), memory.dtype),
        scratch_shapes=[
            pltpu.VMEM((NBUF, B_CH, VAL, MEM), jnp.float32),
            pltpu.VMEM((NBUF, B_CH, VAL, MEM), jnp.float32),
            pltpu.SemaphoreType.DMA((NBUF,)),
            pltpu.SemaphoreType.DMA((NBUF,)),
        ],
    )(mem_t, w, vt)
    return out_t.transpose(0, 2, 1)
